# Initial kernel scaffold; baseline (speedup 1.0000x reference)
import jax
import jax.numpy as jnp
from jax.experimental import pallas as pl


def kernel(x, edge_index, batch, W1, b1, W2, b2, fc_W, fc_b):
    raise NotImplementedError("write your pallas kernel here")


# speedup vs baseline: 29.2395x; 29.2395x over previous
import functools

import jax
import jax.numpy as jnp
from jax import lax
from jax.experimental import pallas as pl
from jax.experimental.pallas import tpu as pltpu
from jax.experimental.pallas import tpu_sc as plsc

N = 10000
E = 320000
D_IN = 128
D_H = 64
NG = 128
NCLS = 10

NC = 2
NS = 16
NW = NC * NS
CH = 100
ROWS = E // CH
RPT = ROWS // NW
NPT = N // NS

CB = 1000
GB = N // CB

_mesh = plsc.VectorSubcoreMesh(core_axis_name="c", subcore_axis_name="s")


def _sc_degree(dst2d, ones, zeros16):
    @functools.partial(
        pl.kernel,
        out_type=jax.ShapeDtypeStruct((NC, NS, NPT, 16), jnp.float32),
        mesh=_mesh,
        compiler_params=pltpu.CompilerParams(use_tc_tiling_on_sc=False),
        scratch_types=[
            pltpu.VMEM((RPT, CH), jnp.int32),
            pltpu.VMEM((CH, 16), jnp.float32),
            pltpu.VMEM_SHARED((N, 16), jnp.float32),
        ],
    )
    def k(dst_hbm, ones_hbm, zeros_hbm, out_hbm, idx_v, ones_v, cnt_s):
        c = lax.axis_index("c")
        s = lax.axis_index("s")
        wid = s * NC + c
        pltpu.sync_copy(zeros_hbm.at[s], cnt_s.at[pl.ds(s * NPT, NPT)])
        pltpu.sync_copy(ones_hbm, ones_v)
        pltpu.sync_copy(dst_hbm.at[wid], idx_v)
        plsc.subcore_barrier()

        @pl.loop(0, RPT)
        def _(j):
            pltpu.sync_copy(ones_v, cnt_s.at[idx_v.at[j]], add=True)

        plsc.subcore_barrier()
        pltpu.sync_copy(cnt_s.at[pl.ds(s * NPT, NPT)], out_hbm.at[c].at[s])

    return k(dst2d, ones, zeros16)


def _sc_scatter(h, src2d, dst2d, zeros64):
    @functools.partial(
        pl.kernel,
        out_type=jax.ShapeDtypeStruct((NC, NS, NPT, D_H), jnp.float32),
        mesh=_mesh,
        compiler_params=pltpu.CompilerParams(use_tc_tiling_on_sc=False),
        scratch_types=[
            pltpu.VMEM((RPT, CH), jnp.int32),
            pltpu.VMEM((RPT, CH), jnp.int32),
            pltpu.VMEM((CH, D_H), jnp.float32),
            pltpu.VMEM_SHARED((N, D_H), jnp.float32),
            pltpu.VMEM_SHARED((N, D_H), jnp.float32),
        ],
    )
    def k(h_hbm, src_hbm, dst_hbm, zeros_hbm, out_hbm,
          src_v, dst_v, rows_v, hs, acc):
        c = lax.axis_index("c")
        s = lax.axis_index("s")
        wid = s * NC + c
        pltpu.sync_copy(h_hbm.at[s], hs.at[pl.ds(s * NPT, NPT)])
        pltpu.sync_copy(zeros_hbm.at[s], acc.at[pl.ds(s * NPT, NPT)])
        pltpu.sync_copy(src_hbm.at[wid], src_v)
        pltpu.sync_copy(dst_hbm.at[wid], dst_v)
        plsc.subcore_barrier()

        @pl.loop(0, RPT)
        def _(j):
            pltpu.sync_copy(hs.at[src_v.at[j]], rows_v)
            pltpu.sync_copy(rows_v, acc.at[dst_v.at[j]], add=True)

        plsc.subcore_barrier()
        pltpu.sync_copy(acc.at[pl.ds(s * NPT, NPT)], out_hbm.at[c].at[s])

    return k(h, src2d, dst2d, zeros64)


def _tc_first(cnt, x, W1):
    def body(cnt_ref, x_ref, w_ref, h_ref, dinv_ref):
        deg = cnt_ref[0, :, 0:1] + cnt_ref[1, :, 0:1] + 1.0
        dinv = lax.rsqrt(deg)
        h = jnp.dot(x_ref[...], w_ref[...], preferred_element_type=jnp.float32)
        h_ref[...] = h * dinv
        dinv_ref[...] = dinv

    return pl.pallas_call(
        body,
        grid=(GB,),
        in_specs=[
            pl.BlockSpec((NC, CB, 16), lambda i: (0, i, 0)),
            pl.BlockSpec((CB, D_IN), lambda i: (i, 0)),
            pl.BlockSpec((D_IN, D_H), lambda i: (0, 0)),
        ],
        out_specs=[
            pl.BlockSpec((CB, D_H), lambda i: (i, 0)),
            pl.BlockSpec((CB, 1), lambda i: (i, 0)),
        ],
        out_shape=[
            jax.ShapeDtypeStruct((N, D_H), jnp.float32),
            jax.ShapeDtypeStruct((N, 1), jnp.float32),
        ],
    )(cnt, x, W1)


def _tc_mid(s1, h1p, dinv, b1, W2):
    def body(s_ref, h_ref, d_ref, b_ref, w_ref, o_ref):
        out1 = (s_ref[0] + s_ref[1] + h_ref[...]) * d_ref[...] + b_ref[...]
        out1 = jnp.maximum(out1, 0.0)
        h2 = jnp.dot(out1, w_ref[...], preferred_element_type=jnp.float32)
        o_ref[...] = h2 * d_ref[...]

    return pl.pallas_call(
        body,
        grid=(GB,),
        in_specs=[
            pl.BlockSpec((NC, CB, D_H), lambda i: (0, i, 0)),
            pl.BlockSpec((CB, D_H), lambda i: (i, 0)),
            pl.BlockSpec((CB, 1), lambda i: (i, 0)),
            pl.BlockSpec((1, D_H), lambda i: (0, 0)),
            pl.BlockSpec((D_H, D_H), lambda i: (0, 0)),
        ],
        out_specs=pl.BlockSpec((CB, D_H), lambda i: (i, 0)),
        out_shape=jax.ShapeDtypeStruct((N, D_H), jnp.float32),
    )(s1, h1p, dinv, b1, W2)


def _tc_final(s2, h2p, dinv, b2, batch3, fc_W, fc_b):
    def body(s_ref, h_ref, d_ref, b_ref, bt_ref, w_ref, fb_ref, o_ref, acc):
        i = pl.program_id(0)

        @pl.when(i == 0)
        def _():
            acc[...] = jnp.zeros_like(acc)

        out2 = (s_ref[0] + s_ref[1] + h_ref[...]) * d_ref[...] + b_ref[...]
        bt = bt_ref[0]
        gids = lax.broadcasted_iota(jnp.int32, (NG, CB), 0)
        onehot = (gids == bt).astype(jnp.float32)
        acc[...] += jnp.dot(onehot, out2, preferred_element_type=jnp.float32)

        @pl.when(i == GB - 1)
        def _():
            logits = jnp.dot(acc[...], w_ref[...],
                             preferred_element_type=jnp.float32) + fb_ref[...]
            m = jnp.max(logits, axis=1, keepdims=True)
            lse = jnp.log(jnp.sum(jnp.exp(logits - m), axis=1,
                                  keepdims=True)) + m
            o_ref[...] = logits - lse

    return pl.pallas_call(
        body,
        grid=(GB,),
        in_specs=[
            pl.BlockSpec((NC, CB, D_H), lambda i: (0, i, 0)),
            pl.BlockSpec((CB, D_H), lambda i: (i, 0)),
            pl.BlockSpec((CB, 1), lambda i: (i, 0)),
            pl.BlockSpec((1, D_H), lambda i: (0, 0)),
            pl.BlockSpec((1, 1, CB), lambda i: (i, 0, 0)),
            pl.BlockSpec((D_H, NCLS), lambda i: (0, 0)),
            pl.BlockSpec((1, NCLS), lambda i: (0, 0)),
        ],
        out_specs=pl.BlockSpec((NG, NCLS), lambda i: (0, 0)),
        out_shape=jax.ShapeDtypeStruct((NG, NCLS), jnp.float32),
        scratch_shapes=[pltpu.VMEM((NG, D_H), jnp.float32)],
    )(s2, h2p, dinv, b2, batch3, fc_W, fc_b)


def kernel(x, edge_index, batch, W1, b1, W2, b2, fc_W, fc_b):
    src3d = edge_index[0].reshape(NW, RPT, CH).astype(jnp.int32)
    dst3d = edge_index[1].reshape(NW, RPT, CH).astype(jnp.int32)
    ones = jnp.ones((CH, 16), jnp.float32)
    zeros16 = jnp.zeros((NS, NPT, 16), jnp.float32)
    zeros64 = jnp.zeros((NS, NPT, D_H), jnp.float32)

    cnt = _sc_degree(dst3d, ones, zeros16).reshape(NC, N, 16)
    h1p, dinv = _tc_first(cnt, x, W1)
    s1 = _sc_scatter(h1p.reshape(NS, NPT, D_H), src3d, dst3d,
                     zeros64).reshape(NC, N, D_H)
    h2p = _tc_mid(s1, h1p, dinv, b1.reshape(1, D_H), W2)
    s2 = _sc_scatter(h2p.reshape(NS, NPT, D_H), src3d, dst3d,
                     zeros64).reshape(NC, N, D_H)
    return _tc_final(s2, h2p, dinv, b2.reshape(1, D_H),
                     batch.reshape(GB, 1, CB).astype(jnp.int32),
                     fc_W, fc_b.reshape(1, NCLS))


# speedup vs baseline: 32.3932x; 1.1079x over previous
import functools

import jax
import jax.numpy as jnp
from jax import lax
from jax.experimental import pallas as pl
from jax.experimental.pallas import tpu as pltpu
from jax.experimental.pallas import tpu_sc as plsc

N = 10000
E = 320000
D_IN = 128
D_H = 64
NG = 128
NCLS = 10

NC = 2
NS = 16
NW = NC * NS
DC = D_H // NC
CH = 100
EPT = E // NS
RPT = EPT // CH
RPW = E // CH // NW
NPT = N // NS

CB = 1000
GB = N // CB

_mesh = plsc.VectorSubcoreMesh(core_axis_name="c", subcore_axis_name="s")
_sc_params = pltpu.CompilerParams(use_tc_tiling_on_sc=False)


def _sc_degree(dst3d, ones, zeros16):
    @functools.partial(
        pl.kernel,
        out_type=jax.ShapeDtypeStruct((NC, NS, NPT, 16), jnp.float32),
        mesh=_mesh,
        compiler_params=_sc_params,
        scratch_types=[
            pltpu.VMEM((RPW, CH), jnp.int32),
            pltpu.VMEM((CH, 16), jnp.float32),
            pltpu.VMEM_SHARED((N, 16), jnp.float32),
            pltpu.SemaphoreType.DMA,
        ],
    )
    def k(dst_hbm, ones_hbm, zeros_hbm, out_hbm, idx_v, ones_v, cnt_s, sem):
        c = lax.axis_index("c")
        s = lax.axis_index("s")
        wid = s * NC + c
        pltpu.sync_copy(zeros_hbm.at[s], cnt_s.at[pl.ds(s * NPT, NPT)])
        pltpu.sync_copy(ones_hbm, ones_v)
        pltpu.sync_copy(dst_hbm.at[wid], idx_v)
        plsc.subcore_barrier()

        K_FIRE = 20

        @pl.loop(0, RPW, step=K_FIRE)
        def _(g):
            for b in range(K_FIRE):
                pltpu.async_copy(ones_v, cnt_s.at[idx_v.at[g + b]], sem,
                                 add=True)
            for b in range(K_FIRE):
                pltpu.make_async_copy(ones_v, cnt_s.at[idx_v.at[g + b]],
                                      sem).wait()

        plsc.subcore_barrier()
        pltpu.sync_copy(cnt_s.at[pl.ds(s * NPT, NPT)], out_hbm.at[c].at[s])

    return k(dst3d, ones, zeros16)


def _sc_scatter(h_split, src3d, dst3d, zeros32):
    NBUF = 5

    @functools.partial(
        pl.kernel,
        out_type=jax.ShapeDtypeStruct((NC, NS, NPT, DC), jnp.float32),
        mesh=_mesh,
        compiler_params=_sc_params,
        scratch_types=[
            pltpu.VMEM((RPT, CH), jnp.int32),
            pltpu.VMEM((RPT, CH), jnp.int32),
            [pltpu.VMEM((CH, DC), jnp.float32)] * NBUF,
            pltpu.VMEM_SHARED((N, DC), jnp.float32),
            pltpu.VMEM_SHARED((N, DC), jnp.float32),
            [pltpu.SemaphoreType.DMA] * NBUF,
            [pltpu.SemaphoreType.DMA] * NBUF,
        ],
    )
    def k(h_hbm, src_hbm, dst_hbm, zeros_hbm, out_hbm,
          src_v, dst_v, rows, hs, acc, semg, sems):
        c = lax.axis_index("c")
        s = lax.axis_index("s")
        pltpu.sync_copy(h_hbm.at[c].at[s], hs.at[pl.ds(s * NPT, NPT)])
        pltpu.sync_copy(zeros_hbm.at[s], acc.at[pl.ds(s * NPT, NPT)])
        pltpu.sync_copy(src_hbm.at[s], src_v)
        pltpu.sync_copy(dst_hbm.at[s], dst_v)
        plsc.subcore_barrier()

        @pl.loop(0, RPT, step=NBUF)
        def _(g):
            for b in range(NBUF):
                @pl.when(g > 0)
                def _():
                    pltpu.make_async_copy(
                        rows[b], acc.at[dst_v.at[g + b - NBUF]],
                        sems[b]).wait()
                pltpu.async_copy(hs.at[src_v.at[g + b]], rows[b], semg[b])
            for b in range(NBUF):
                pltpu.make_async_copy(hs.at[src_v.at[g + b]], rows[b],
                                      semg[b]).wait()
                pltpu.async_copy(rows[b], acc.at[dst_v.at[g + b]], sems[b],
                                 add=True)

        for b in range(NBUF):
            pltpu.make_async_copy(rows[b], acc.at[dst_v.at[RPT - NBUF + b]],
                                  sems[b]).wait()

        plsc.subcore_barrier()
        pltpu.sync_copy(acc.at[pl.ds(s * NPT, NPT)], out_hbm.at[c].at[s])

    return k(h_split, src3d, dst3d, zeros32)


def _tc_first(cnt, x, W1):
    def body(cnt_ref, x_ref, w_ref, h_ref, dinv_ref):
        deg = cnt_ref[0, :, 0:1] + cnt_ref[1, :, 0:1] + 1.0
        dinv = lax.rsqrt(deg)
        h = jnp.dot(x_ref[...], w_ref[...],
                    preferred_element_type=jnp.float32) * dinv
        h_ref[0] = h[:, :DC]
        h_ref[1] = h[:, DC:]
        dinv_ref[...] = dinv

    return pl.pallas_call(
        body,
        grid=(GB,),
        in_specs=[
            pl.BlockSpec((NC, CB, 16), lambda i: (0, i, 0)),
            pl.BlockSpec((CB, D_IN), lambda i: (i, 0)),
            pl.BlockSpec((D_IN, D_H), lambda i: (0, 0)),
        ],
        out_specs=[
            pl.BlockSpec((NC, CB, DC), lambda i: (0, i, 0)),
            pl.BlockSpec((CB, 1), lambda i: (i, 0)),
        ],
        out_shape=[
            jax.ShapeDtypeStruct((NC, N, DC), jnp.float32),
            jax.ShapeDtypeStruct((N, 1), jnp.float32),
        ],
    )(cnt, x, W1)


def _tc_mid(s1, h1p, dinv, b1, W2):
    def body(s_ref, h_ref, d_ref, b_ref, w_ref, o_ref):
        t = jnp.concatenate([s_ref[0] + h_ref[0], s_ref[1] + h_ref[1]],
                            axis=1)
        out1 = jnp.maximum(t * d_ref[...] + b_ref[...], 0.0)
        h2 = jnp.dot(out1, w_ref[...],
                     preferred_element_type=jnp.float32) * d_ref[...]
        o_ref[0] = h2[:, :DC]
        o_ref[1] = h2[:, DC:]

    return pl.pallas_call(
        body,
        grid=(GB,),
        in_specs=[
            pl.BlockSpec((NC, CB, DC), lambda i: (0, i, 0)),
            pl.BlockSpec((NC, CB, DC), lambda i: (0, i, 0)),
            pl.BlockSpec((CB, 1), lambda i: (i, 0)),
            pl.BlockSpec((1, D_H), lambda i: (0, 0)),
            pl.BlockSpec((D_H, D_H), lambda i: (0, 0)),
        ],
        out_specs=pl.BlockSpec((NC, CB, DC), lambda i: (0, i, 0)),
        out_shape=jax.ShapeDtypeStruct((NC, N, DC), jnp.float32),
    )(s1, h1p, dinv, b1, W2)


def _tc_final(s2, h2p, dinv, b2, batch3, fc_W, fc_b):
    def body(s_ref, h_ref, d_ref, b_ref, bt_ref, w_ref, fb_ref, o_ref, acc):
        i = pl.program_id(0)

        @pl.when(i == 0)
        def _():
            acc[...] = jnp.zeros_like(acc)

        t = jnp.concatenate([s_ref[0] + h_ref[0], s_ref[1] + h_ref[1]],
                            axis=1)
        out2 = t * d_ref[...] + b_ref[...]
        bt = bt_ref[0]
        gids = lax.broadcasted_iota(jnp.int32, (NG, CB), 0)
        onehot = (gids == bt).astype(jnp.float32)
        acc[...] += jnp.dot(onehot, out2, preferred_element_type=jnp.float32)

        @pl.when(i == GB - 1)
        def _():
            logits = jnp.dot(acc[...], w_ref[...],
                             preferred_element_type=jnp.float32) + fb_ref[...]
            m = jnp.max(logits, axis=1, keepdims=True)
            lse = jnp.log(jnp.sum(jnp.exp(logits - m), axis=1,
                                  keepdims=True)) + m
            o_ref[...] = logits - lse

    return pl.pallas_call(
        body,
        grid=(GB,),
        in_specs=[
            pl.BlockSpec((NC, CB, DC), lambda i: (0, i, 0)),
            pl.BlockSpec((NC, CB, DC), lambda i: (0, i, 0)),
            pl.BlockSpec((CB, 1), lambda i: (i, 0)),
            pl.BlockSpec((1, D_H), lambda i: (0, 0)),
            pl.BlockSpec((1, 1, CB), lambda i: (i, 0, 0)),
            pl.BlockSpec((D_H, NCLS), lambda i: (0, 0)),
            pl.BlockSpec((1, NCLS), lambda i: (0, 0)),
        ],
        out_specs=pl.BlockSpec((NG, NCLS), lambda i: (0, 0)),
        out_shape=jax.ShapeDtypeStruct((NG, NCLS), jnp.float32),
        scratch_shapes=[pltpu.VMEM((NG, D_H), jnp.float32)],
    )(s2, h2p, dinv, b2, batch3, fc_W, fc_b)


def kernel(x, edge_index, batch, W1, b1, W2, b2, fc_W, fc_b):
    src = edge_index[0].astype(jnp.int32)
    dst = edge_index[1].astype(jnp.int32)
    src3d = src.reshape(NS, RPT, CH)
    dst3d = dst.reshape(NS, RPT, CH)
    dstw = dst.reshape(NW, RPW, CH)
    ones = jnp.ones((CH, 16), jnp.float32)
    zeros16 = jnp.zeros((NS, NPT, 16), jnp.float32)
    zeros32 = jnp.zeros((NS, NPT, DC), jnp.float32)

    cnt = _sc_degree(dstw, ones, zeros16).reshape(NC, N, 16)
    h1p, dinv = _tc_first(cnt, x, W1)
    s1 = _sc_scatter(h1p.reshape(NC, NS, NPT, DC), src3d, dst3d,
                     zeros32).reshape(NC, N, DC)
    h2p = _tc_mid(s1, h1p, dinv, b1.reshape(1, D_H), W2)
    s2 = _sc_scatter(h2p.reshape(NC, NS, NPT, DC), src3d, dst3d,
                     zeros32).reshape(NC, N, DC)
    return _tc_final(s2, h2p, dinv, b2.reshape(1, D_H),
                     batch.reshape(GB, 1, CB).astype(jnp.int32),
                     fc_W, fc_b.reshape(1, NCLS))


# speedup vs baseline: 33.2867x; 1.0276x over previous
import functools

import jax
import jax.numpy as jnp
from jax import lax
from jax.experimental import pallas as pl
from jax.experimental.pallas import tpu as pltpu
from jax.experimental.pallas import tpu_sc as plsc

N = 10000
E = 320000
D_IN = 128
D_H = 64
NG = 128
NCLS = 10

NC = 2
NS = 16
NW = NC * NS
DC = D_H // NC
CH = 125
EPT = E // NS
RPT = EPT // CH
RPW = E // CH // NW
NPT = N // NS

CB = 1000
GB = N // CB

_mesh = plsc.VectorSubcoreMesh(core_axis_name="c", subcore_axis_name="s")
_sc_params = pltpu.CompilerParams(use_tc_tiling_on_sc=False)


def _sc_degree(dst3d, ones, zeros16):
    @functools.partial(
        pl.kernel,
        out_type=jax.ShapeDtypeStruct((NC, NS, NPT, 16), jnp.float32),
        mesh=_mesh,
        compiler_params=_sc_params,
        scratch_types=[
            pltpu.VMEM((RPW, CH), jnp.int32),
            pltpu.VMEM((CH, 16), jnp.float32),
            pltpu.VMEM_SHARED((N, 16), jnp.float32),
            pltpu.SemaphoreType.DMA,
        ],
    )
    def k(dst_hbm, ones_hbm, zeros_hbm, out_hbm, idx_v, ones_v, cnt_s, sem):
        c = lax.axis_index("c")
        s = lax.axis_index("s")
        wid = s * NC + c
        pltpu.sync_copy(zeros_hbm.at[s], cnt_s.at[pl.ds(s * NPT, NPT)])
        pltpu.sync_copy(ones_hbm, ones_v)
        pltpu.sync_copy(dst_hbm.at[wid], idx_v)
        plsc.subcore_barrier()

        K_FIRE = 20

        @pl.loop(0, RPW, step=K_FIRE)
        def _(g):
            for b in range(K_FIRE):
                pltpu.async_copy(ones_v, cnt_s.at[idx_v.at[g + b]], sem,
                                 add=True)
            for b in range(K_FIRE):
                pltpu.make_async_copy(ones_v, cnt_s.at[idx_v.at[g + b]],
                                      sem).wait()

        plsc.subcore_barrier()
        pltpu.sync_copy(cnt_s.at[pl.ds(s * NPT, NPT)], out_hbm.at[c].at[s])

    return k(dst3d, ones, zeros16)


def _sc_scatter(h_split, src3d, dst3d, zeros32):
    NBUF = 8

    @functools.partial(
        pl.kernel,
        out_type=jax.ShapeDtypeStruct((NC, NS, NPT, DC), jnp.float32),
        mesh=_mesh,
        compiler_params=_sc_params,
        scratch_types=[
            pltpu.VMEM((RPT, CH), jnp.int32),
            pltpu.VMEM((RPT, CH), jnp.int32),
            [pltpu.VMEM((CH, DC), jnp.float32)] * NBUF,
            pltpu.VMEM_SHARED((N, DC), jnp.float32),
            pltpu.VMEM_SHARED((N, DC), jnp.float32),
            [pltpu.SemaphoreType.DMA] * NBUF,
            [pltpu.SemaphoreType.DMA] * NBUF,
        ],
    )
    def k(h_hbm, src_hbm, dst_hbm, zeros_hbm, out_hbm,
          src_v, dst_v, rows, hs, acc, semg, sems):
        c = lax.axis_index("c")
        s = lax.axis_index("s")
        pltpu.sync_copy(h_hbm.at[c].at[s], hs.at[pl.ds(s * NPT, NPT)])
        pltpu.sync_copy(zeros_hbm.at[s], acc.at[pl.ds(s * NPT, NPT)])
        pltpu.sync_copy(src_hbm.at[s], src_v)
        pltpu.sync_copy(dst_hbm.at[s], dst_v)
        plsc.subcore_barrier()

        @pl.loop(0, RPT, step=NBUF)
        def _(g):
            for b in range(NBUF):
                @pl.when(g > 0)
                def _():
                    pltpu.make_async_copy(
                        rows[b], acc.at[dst_v.at[g + b - NBUF]],
                        sems[b]).wait()
                pltpu.async_copy(hs.at[src_v.at[g + b]], rows[b], semg[b])
            for b in range(NBUF):
                pltpu.make_async_copy(hs.at[src_v.at[g + b]], rows[b],
                                      semg[b]).wait()
                pltpu.async_copy(rows[b], acc.at[dst_v.at[g + b]], sems[b],
                                 add=True)

        for b in range(NBUF):
            pltpu.make_async_copy(rows[b], acc.at[dst_v.at[RPT - NBUF + b]],
                                  sems[b]).wait()

        plsc.subcore_barrier()
        pltpu.sync_copy(acc.at[pl.ds(s * NPT, NPT)], out_hbm.at[c].at[s])

    return k(h_split, src3d, dst3d, zeros32)


def _tc_first(cnt, x, W1):
    def body(cnt_ref, x_ref, w_ref, h_ref, dinv_ref):
        deg = cnt_ref[0, :, 0:1] + cnt_ref[1, :, 0:1] + 1.0
        dinv = lax.rsqrt(deg)
        h = jnp.dot(x_ref[...], w_ref[...],
                    preferred_element_type=jnp.float32) * dinv
        h_ref[0] = h[:, :DC]
        h_ref[1] = h[:, DC:]
        dinv_ref[...] = dinv

    return pl.pallas_call(
        body,
        grid=(GB,),
        in_specs=[
            pl.BlockSpec((NC, CB, 16), lambda i: (0, i, 0)),
            pl.BlockSpec((CB, D_IN), lambda i: (i, 0)),
            pl.BlockSpec((D_IN, D_H), lambda i: (0, 0)),
        ],
        out_specs=[
            pl.BlockSpec((NC, CB, DC), lambda i: (0, i, 0)),
            pl.BlockSpec((CB, 1), lambda i: (i, 0)),
        ],
        out_shape=[
            jax.ShapeDtypeStruct((NC, N, DC), jnp.float32),
            jax.ShapeDtypeStruct((N, 1), jnp.float32),
        ],
    )(cnt, x, W1)


def _tc_mid(s1, h1p, dinv, b1, W2):
    def body(s_ref, h_ref, d_ref, b_ref, w_ref, o_ref):
        t = jnp.concatenate([s_ref[0] + h_ref[0], s_ref[1] + h_ref[1]],
                            axis=1)
        out1 = jnp.maximum(t * d_ref[...] + b_ref[...], 0.0)
        h2 = jnp.dot(out1, w_ref[...],
                     preferred_element_type=jnp.float32) * d_ref[...]
        o_ref[0] = h2[:, :DC]
        o_ref[1] = h2[:, DC:]

    return pl.pallas_call(
        body,
        grid=(GB,),
        in_specs=[
            pl.BlockSpec((NC, CB, DC), lambda i: (0, i, 0)),
            pl.BlockSpec((NC, CB, DC), lambda i: (0, i, 0)),
            pl.BlockSpec((CB, 1), lambda i: (i, 0)),
            pl.BlockSpec((1, D_H), lambda i: (0, 0)),
            pl.BlockSpec((D_H, D_H), lambda i: (0, 0)),
        ],
        out_specs=pl.BlockSpec((NC, CB, DC), lambda i: (0, i, 0)),
        out_shape=jax.ShapeDtypeStruct((NC, N, DC), jnp.float32),
    )(s1, h1p, dinv, b1, W2)


def _tc_final(s2, h2p, dinv, b2, batch3, fc_W, fc_b):
    def body(s_ref, h_ref, d_ref, b_ref, bt_ref, w_ref, fb_ref, o_ref, acc):
        i = pl.program_id(0)

        @pl.when(i == 0)
        def _():
            acc[...] = jnp.zeros_like(acc)

        t = jnp.concatenate([s_ref[0] + h_ref[0], s_ref[1] + h_ref[1]],
                            axis=1)
        out2 = t * d_ref[...] + b_ref[...]
        bt = bt_ref[0]
        gids = lax.broadcasted_iota(jnp.int32, (NG, CB), 0)
        onehot = (gids == bt).astype(jnp.float32)
        acc[...] += jnp.dot(onehot, out2, preferred_element_type=jnp.float32)

        @pl.when(i == GB - 1)
        def _():
            logits = jnp.dot(acc[...], w_ref[...],
                             preferred_element_type=jnp.float32) + fb_ref[...]
            m = jnp.max(logits, axis=1, keepdims=True)
            lse = jnp.log(jnp.sum(jnp.exp(logits - m), axis=1,
                                  keepdims=True)) + m
            o_ref[...] = logits - lse

    return pl.pallas_call(
        body,
        grid=(GB,),
        in_specs=[
            pl.BlockSpec((NC, CB, DC), lambda i: (0, i, 0)),
            pl.BlockSpec((NC, CB, DC), lambda i: (0, i, 0)),
            pl.BlockSpec((CB, 1), lambda i: (i, 0)),
            pl.BlockSpec((1, D_H), lambda i: (0, 0)),
            pl.BlockSpec((1, 1, CB), lambda i: (i, 0, 0)),
            pl.BlockSpec((D_H, NCLS), lambda i: (0, 0)),
            pl.BlockSpec((1, NCLS), lambda i: (0, 0)),
        ],
        out_specs=pl.BlockSpec((NG, NCLS), lambda i: (0, 0)),
        out_shape=jax.ShapeDtypeStruct((NG, NCLS), jnp.float32),
        scratch_shapes=[pltpu.VMEM((NG, D_H), jnp.float32)],
    )(s2, h2p, dinv, b2, batch3, fc_W, fc_b)


def kernel(x, edge_index, batch, W1, b1, W2, b2, fc_W, fc_b):
    src = edge_index[0].astype(jnp.int32)
    dst = edge_index[1].astype(jnp.int32)
    src3d = src.reshape(NS, RPT, CH)
    dst3d = dst.reshape(NS, RPT, CH)
    dstw = dst.reshape(NW, RPW, CH)
    ones = jnp.ones((CH, 16), jnp.float32)
    zeros16 = jnp.zeros((NS, NPT, 16), jnp.float32)
    zeros32 = jnp.zeros((NS, NPT, DC), jnp.float32)

    cnt = _sc_degree(dstw, ones, zeros16).reshape(NC, N, 16)
    h1p, dinv = _tc_first(cnt, x, W1)
    s1 = _sc_scatter(h1p.reshape(NC, NS, NPT, DC), src3d, dst3d,
                     zeros32).reshape(NC, N, DC)
    h2p = _tc_mid(s1, h1p, dinv, b1.reshape(1, D_H), W2)
    s2 = _sc_scatter(h2p.reshape(NC, NS, NPT, DC), src3d, dst3d,
                     zeros32).reshape(NC, N, DC)
    return _tc_final(s2, h2p, dinv, b2.reshape(1, D_H),
                     batch.reshape(GB, 1, CB).astype(jnp.int32),
                     fc_W, fc_b.reshape(1, NCLS))


# speedup vs baseline: 33.3240x; 1.0011x over previous
import functools

import jax
import jax.numpy as jnp
from jax import lax
from jax.experimental import pallas as pl
from jax.experimental.pallas import tpu as pltpu
from jax.experimental.pallas import tpu_sc as plsc

N = 10000
E = 320000
D_IN = 128
D_H = 64
NG = 128
NCLS = 10

NC = 2
NS = 16
NW = NC * NS
DC = D_H // NC
CH = 125
EPT = E // NS
RPT = EPT // CH
RPW = E // CH // NW
NPT = N // NS

CB = 1000
GB = N // CB

_mesh = plsc.VectorSubcoreMesh(core_axis_name="c", subcore_axis_name="s")
_sc_params = pltpu.CompilerParams(use_tc_tiling_on_sc=False)


def _sc_degree(dst3d, ones, zeros16):
    @functools.partial(
        pl.kernel,
        out_type=jax.ShapeDtypeStruct((NC, NS, NPT, 16), jnp.float32),
        mesh=_mesh,
        compiler_params=_sc_params,
        scratch_types=[
            pltpu.VMEM((RPW, CH), jnp.int32),
            pltpu.VMEM((CH, 16), jnp.float32),
            pltpu.VMEM_SHARED((N, 16), jnp.float32),
            pltpu.SemaphoreType.DMA,
        ],
    )
    def k(dst_hbm, ones_hbm, zeros_hbm, out_hbm, idx_v, ones_v, cnt_s, sem):
        c = lax.axis_index("c")
        s = lax.axis_index("s")
        wid = s * NC + c
        p0 = pltpu.async_copy(zeros_hbm.at[s],
                              cnt_s.at[pl.ds(s * NPT, NPT)], sem)
        p1 = pltpu.async_copy(ones_hbm, ones_v, sem)
        p2 = pltpu.async_copy(dst_hbm.at[wid], idx_v, sem)
        p0.wait()
        p1.wait()
        p2.wait()
        plsc.subcore_barrier()

        K_FIRE = 20

        @pl.loop(0, RPW, step=K_FIRE)
        def _(g):
            for b in range(K_FIRE):
                pltpu.async_copy(ones_v, cnt_s.at[idx_v.at[g + b]], sem,
                                 add=True)
            for b in range(K_FIRE):
                pltpu.make_async_copy(ones_v, cnt_s.at[idx_v.at[g + b]],
                                      sem).wait()

        plsc.subcore_barrier()
        pltpu.sync_copy(cnt_s.at[pl.ds(s * NPT, NPT)], out_hbm.at[c].at[s])

    return k(dst3d, ones, zeros16)


def _sc_scatter(h_split, src3d, dst3d, zeros32):
    NBUF = 8

    @functools.partial(
        pl.kernel,
        out_type=jax.ShapeDtypeStruct((NC, NS, NPT, DC), jnp.float32),
        mesh=_mesh,
        compiler_params=_sc_params,
        scratch_types=[
            pltpu.VMEM((RPT, CH), jnp.int32),
            pltpu.VMEM((RPT, CH), jnp.int32),
            [pltpu.VMEM((CH, DC), jnp.float32)] * NBUF,
            pltpu.VMEM_SHARED((N, DC), jnp.float32),
            pltpu.VMEM_SHARED((N, DC), jnp.float32),
            [pltpu.SemaphoreType.DMA] * NBUF,
            [pltpu.SemaphoreType.DMA] * NBUF,
        ],
    )
    def k(h_hbm, src_hbm, dst_hbm, zeros_hbm, out_hbm,
          src_v, dst_v, rows, hs, acc, semg, sems):
        c = lax.axis_index("c")
        s = lax.axis_index("s")
        p0 = pltpu.async_copy(h_hbm.at[c].at[s],
                              hs.at[pl.ds(s * NPT, NPT)], semg[0])
        p1 = pltpu.async_copy(zeros_hbm.at[s],
                              acc.at[pl.ds(s * NPT, NPT)], semg[1])
        p2 = pltpu.async_copy(src_hbm.at[s], src_v, semg[2])
        p3 = pltpu.async_copy(dst_hbm.at[s], dst_v, semg[3])
        p0.wait()
        p1.wait()
        p2.wait()
        p3.wait()
        plsc.subcore_barrier()

        @pl.loop(0, RPT, step=NBUF)
        def _(g):
            for b in range(NBUF):
                @pl.when(g > 0)
                def _():
                    pltpu.make_async_copy(
                        rows[b], acc.at[dst_v.at[g + b - NBUF]],
                        sems[b]).wait()
                pltpu.async_copy(hs.at[src_v.at[g + b]], rows[b], semg[b])
            for b in range(NBUF):
                pltpu.make_async_copy(hs.at[src_v.at[g + b]], rows[b],
                                      semg[b]).wait()
                pltpu.async_copy(rows[b], acc.at[dst_v.at[g + b]], sems[b],
                                 add=True)

        for b in range(NBUF):
            pltpu.make_async_copy(rows[b], acc.at[dst_v.at[RPT - NBUF + b]],
                                  sems[b]).wait()

        plsc.subcore_barrier()
        pltpu.sync_copy(acc.at[pl.ds(s * NPT, NPT)], out_hbm.at[c].at[s])

    return k(h_split, src3d, dst3d, zeros32)


def _tc_mm(x, W1):
    def body(x_ref, w_ref, h_ref):
        h_ref[...] = jnp.dot(x_ref[...], w_ref[...],
                             preferred_element_type=jnp.float32)

    return pl.pallas_call(
        body,
        grid=(GB,),
        in_specs=[
            pl.BlockSpec((CB, D_IN), lambda i: (i, 0)),
            pl.BlockSpec((D_IN, D_H), lambda i: (0, 0)),
        ],
        out_specs=pl.BlockSpec((CB, D_H), lambda i: (i, 0)),
        out_shape=jax.ShapeDtypeStruct((N, D_H), jnp.float32),
    )(x, W1)


def _tc_scale(cnt, h1):
    def body(cnt_ref, h1_ref, h_ref, dinv_ref):
        deg = cnt_ref[0, :, 0:1] + cnt_ref[1, :, 0:1] + 1.0
        dinv = lax.rsqrt(deg)
        h = h1_ref[...] * dinv
        h_ref[0] = h[:, :DC]
        h_ref[1] = h[:, DC:]
        dinv_ref[...] = dinv

    return pl.pallas_call(
        body,
        grid=(GB,),
        in_specs=[
            pl.BlockSpec((NC, CB, 16), lambda i: (0, i, 0)),
            pl.BlockSpec((CB, D_H), lambda i: (i, 0)),
        ],
        out_specs=[
            pl.BlockSpec((NC, CB, DC), lambda i: (0, i, 0)),
            pl.BlockSpec((CB, 1), lambda i: (i, 0)),
        ],
        out_shape=[
            jax.ShapeDtypeStruct((NC, N, DC), jnp.float32),
            jax.ShapeDtypeStruct((N, 1), jnp.float32),
        ],
    )(cnt, h1)


def _tc_mid(s1, h1p, dinv, b1, W2):
    def body(s_ref, h_ref, d_ref, b_ref, w_ref, o_ref):
        t = jnp.concatenate([s_ref[0] + h_ref[0], s_ref[1] + h_ref[1]],
                            axis=1)
        out1 = jnp.maximum(t * d_ref[...] + b_ref[...], 0.0)
        h2 = jnp.dot(out1, w_ref[...],
                     preferred_element_type=jnp.float32) * d_ref[...]
        o_ref[0] = h2[:, :DC]
        o_ref[1] = h2[:, DC:]

    return pl.pallas_call(
        body,
        grid=(GB,),
        in_specs=[
            pl.BlockSpec((NC, CB, DC), lambda i: (0, i, 0)),
            pl.BlockSpec((NC, CB, DC), lambda i: (0, i, 0)),
            pl.BlockSpec((CB, 1), lambda i: (i, 0)),
            pl.BlockSpec((1, D_H), lambda i: (0, 0)),
            pl.BlockSpec((D_H, D_H), lambda i: (0, 0)),
        ],
        out_specs=pl.BlockSpec((NC, CB, DC), lambda i: (0, i, 0)),
        out_shape=jax.ShapeDtypeStruct((NC, N, DC), jnp.float32),
    )(s1, h1p, dinv, b1, W2)


def _tc_final(s2, h2p, dinv, b2, batch3, fc_W, fc_b):
    def body(s_ref, h_ref, d_ref, b_ref, bt_ref, w_ref, fb_ref, o_ref, acc):
        i = pl.program_id(0)

        @pl.when(i == 0)
        def _():
            acc[...] = jnp.zeros_like(acc)

        t = jnp.concatenate([s_ref[0] + h_ref[0], s_ref[1] + h_ref[1]],
                            axis=1)
        out2 = t * d_ref[...] + b_ref[...]
        bt = bt_ref[0]
        gids = lax.broadcasted_iota(jnp.int32, (NG, CB), 0)
        onehot = (gids == bt).astype(jnp.float32)
        acc[...] += jnp.dot(onehot, out2, preferred_element_type=jnp.float32)

        @pl.when(i == GB - 1)
        def _():
            logits = jnp.dot(acc[...], w_ref[...],
                             preferred_element_type=jnp.float32) + fb_ref[...]
            m = jnp.max(logits, axis=1, keepdims=True)
            lse = jnp.log(jnp.sum(jnp.exp(logits - m), axis=1,
                                  keepdims=True)) + m
            o_ref[...] = logits - lse

    return pl.pallas_call(
        body,
        grid=(GB,),
        in_specs=[
            pl.BlockSpec((NC, CB, DC), lambda i: (0, i, 0)),
            pl.BlockSpec((NC, CB, DC), lambda i: (0, i, 0)),
            pl.BlockSpec((CB, 1), lambda i: (i, 0)),
            pl.BlockSpec((1, D_H), lambda i: (0, 0)),
            pl.BlockSpec((1, 1, CB), lambda i: (i, 0, 0)),
            pl.BlockSpec((D_H, NCLS), lambda i: (0, 0)),
            pl.BlockSpec((1, NCLS), lambda i: (0, 0)),
        ],
        out_specs=pl.BlockSpec((NG, NCLS), lambda i: (0, 0)),
        out_shape=jax.ShapeDtypeStruct((NG, NCLS), jnp.float32),
        scratch_shapes=[pltpu.VMEM((NG, D_H), jnp.float32)],
    )(s2, h2p, dinv, b2, batch3, fc_W, fc_b)


def kernel(x, edge_index, batch, W1, b1, W2, b2, fc_W, fc_b):
    src = edge_index[0].astype(jnp.int32)
    dst = edge_index[1].astype(jnp.int32)
    src3d = src.reshape(NS, RPT, CH)
    dst3d = dst.reshape(NS, RPT, CH)
    dstw = dst.reshape(NW, RPW, CH)
    ones = jnp.ones((CH, 16), jnp.float32)
    zeros16 = jnp.zeros((NS, NPT, 16), jnp.float32)
    zeros32 = jnp.zeros((NS, NPT, DC), jnp.float32)

    h1 = _tc_mm(x, W1)
    cnt = _sc_degree(dstw, ones, zeros16).reshape(NC, N, 16)
    h1p, dinv = _tc_scale(cnt, h1)
    s1 = _sc_scatter(h1p.reshape(NC, NS, NPT, DC), src3d, dst3d,
                     zeros32).reshape(NC, N, DC)
    h2p = _tc_mid(s1, h1p, dinv, b1.reshape(1, D_H), W2)
    s2 = _sc_scatter(h2p.reshape(NC, NS, NPT, DC), src3d, dst3d,
                     zeros32).reshape(NC, N, DC)
    return _tc_final(s2, h2p, dinv, b2.reshape(1, D_H),
                     batch.reshape(GB, 1, CB).astype(jnp.int32),
                     fc_W, fc_b.reshape(1, NCLS))


# speedup vs baseline: 38.4446x; 1.1537x over previous
import functools

import jax
import jax.numpy as jnp
from jax import lax
from jax.experimental import pallas as pl
from jax.experimental.pallas import tpu as pltpu
from jax.experimental.pallas import tpu_sc as plsc

N = 10000
E = 320000
D_IN = 128
D_H = 64
NG = 128
NCLS = 10
LANES = 128

NC = 2
NS = 16
NW = NC * NS
DC = D_H // NC
CH = 80
EPT = E // NS
RPT = EPT // CH
EPW = E // NW
RPW = EPW // CH
NPT = N // NS

CB = 1000
GB = N // CB

_mesh = plsc.VectorSubcoreMesh(core_axis_name="c", subcore_axis_name="s")
_sc_params = pltpu.CompilerParams(use_tc_tiling_on_sc=False)


def _sc_degree(dst, ones, zeros):
    @functools.partial(
        pl.kernel,
        out_type=jax.ShapeDtypeStruct((N, LANES), jnp.float32),
        mesh=_mesh,
        compiler_params=_sc_params,
        scratch_types=[
            pltpu.VMEM((EPW,), jnp.int32),
            pltpu.VMEM((CH, 16), jnp.float32),
            pltpu.VMEM_SHARED((N, 16), jnp.float32),
            pltpu.SemaphoreType.DMA,
            pltpu.SemaphoreType.DMA,
        ],
    )
    def k(dst_hbm, ones_hbm, zeros_hbm, out_hbm, idx_v, ones_v, cnt_s,
          sem, semp):
        c = lax.axis_index("c")
        s = lax.axis_index("s")
        wid = s * NC + c
        p0 = pltpu.async_copy(zeros_hbm.at[pl.ds(s * NPT, NPT), pl.ds(0, 16)],
                              cnt_s.at[pl.ds(s * NPT, NPT)], semp)
        p1 = pltpu.async_copy(ones_hbm, ones_v, semp)
        p2 = pltpu.async_copy(dst_hbm.at[pl.ds(wid * EPW, EPW)], idx_v, sem)
        p0.wait()
        p1.wait()
        p2.wait()
        plsc.subcore_barrier()

        K_FIRE = 25

        @pl.loop(0, RPW, step=K_FIRE)
        def _(g):
            for b in range(K_FIRE):
                pltpu.async_copy(
                    ones_v, cnt_s.at[idx_v.at[pl.ds((g + b) * CH, CH)]],
                    sem, add=True)
            for b in range(K_FIRE):
                pltpu.make_async_copy(
                    ones_v, cnt_s.at[idx_v.at[pl.ds((g + b) * CH, CH)]],
                    sem).wait()

        plsc.subcore_barrier()
        pltpu.sync_copy(cnt_s.at[pl.ds(s * NPT, NPT)],
                        out_hbm.at[pl.ds(s * NPT, NPT), pl.ds(c * 16, 16)])

    return k(dst, ones, zeros)


def _sc_scatter(h, src, dst, zeros):
    NBUF = 10

    @functools.partial(
        pl.kernel,
        out_type=jax.ShapeDtypeStruct((N, LANES), jnp.float32),
        mesh=_mesh,
        compiler_params=_sc_params,
        scratch_types=[
            pltpu.VMEM((EPT,), jnp.int32),
            pltpu.VMEM((EPT,), jnp.int32),
            [pltpu.VMEM((CH, DC), jnp.float32)] * NBUF,
            pltpu.VMEM_SHARED((N, DC), jnp.float32),
            pltpu.VMEM_SHARED((N, DC), jnp.float32),
            [pltpu.SemaphoreType.DMA] * NBUF,
            [pltpu.SemaphoreType.DMA] * NBUF,
        ],
    )
    def k(h_hbm, src_hbm, dst_hbm, zeros_hbm, out_hbm,
          src_v, dst_v, rows, hs, acc, semg, sems):
        c = lax.axis_index("c")
        s = lax.axis_index("s")
        p0 = pltpu.async_copy(
            h_hbm.at[pl.ds(s * NPT, NPT), pl.ds(c * DC, DC)],
            hs.at[pl.ds(s * NPT, NPT)], semg[0])
        p1 = pltpu.async_copy(
            zeros_hbm.at[pl.ds(s * NPT, NPT), pl.ds(0, DC)],
            acc.at[pl.ds(s * NPT, NPT)], semg[1])
        p2 = pltpu.async_copy(src_hbm.at[pl.ds(s * EPT, EPT)], src_v, semg[2])
        p3 = pltpu.async_copy(dst_hbm.at[pl.ds(s * EPT, EPT)], dst_v, semg[3])
        p0.wait()
        p1.wait()
        p2.wait()
        p3.wait()
        plsc.subcore_barrier()

        @pl.loop(0, RPT, step=NBUF)
        def _(g):
            for b in range(NBUF):
                @pl.when(g > 0)
                def _():
                    pltpu.make_async_copy(
                        rows[b],
                        acc.at[dst_v.at[pl.ds((g + b - NBUF) * CH, CH)]],
                        sems[b]).wait()
                pltpu.async_copy(
                    hs.at[src_v.at[pl.ds((g + b) * CH, CH)]], rows[b],
                    semg[b])
            for b in range(NBUF):
                pltpu.make_async_copy(
                    hs.at[src_v.at[pl.ds((g + b) * CH, CH)]], rows[b],
                    semg[b]).wait()
                pltpu.async_copy(
                    rows[b], acc.at[dst_v.at[pl.ds((g + b) * CH, CH)]],
                    sems[b], add=True)

        for b in range(NBUF):
            pltpu.make_async_copy(
                rows[b], acc.at[dst_v.at[pl.ds((RPT - NBUF + b) * CH, CH)]],
                sems[b]).wait()

        plsc.subcore_barrier()
        pltpu.sync_copy(acc.at[pl.ds(s * NPT, NPT)],
                        out_hbm.at[pl.ds(s * NPT, NPT), pl.ds(c * DC, DC)])

    return k(h, src, dst, zeros)


def _tc_mm(x, W1):
    def body(x_ref, w_ref, h_ref):
        h_ref[...] = jnp.dot(x_ref[...], w_ref[...],
                             preferred_element_type=jnp.float32)

    return pl.pallas_call(
        body,
        grid=(GB,),
        in_specs=[
            pl.BlockSpec((CB, D_IN), lambda i: (i, 0)),
            pl.BlockSpec((D_IN, D_H), lambda i: (0, 0)),
        ],
        out_specs=pl.BlockSpec((CB, D_H), lambda i: (i, 0)),
        out_shape=jax.ShapeDtypeStruct((N, D_H), jnp.float32),
    )(x, W1)


def _tc_scale(cnt, h1):
    def body(cnt_ref, h1_ref, h_ref, dinv_ref):
        deg = cnt_ref[:, 0:1] + cnt_ref[:, 16:17] + 1.0
        dinv = lax.rsqrt(deg)
        h = h1_ref[...] * dinv
        h_ref[...] = jnp.concatenate([h, jnp.zeros_like(h)], axis=1)
        dinv_ref[...] = dinv

    return pl.pallas_call(
        body,
        grid=(GB,),
        in_specs=[
            pl.BlockSpec((CB, LANES), lambda i: (i, 0)),
            pl.BlockSpec((CB, D_H), lambda i: (i, 0)),
        ],
        out_specs=[
            pl.BlockSpec((CB, LANES), lambda i: (i, 0)),
            pl.BlockSpec((CB, 1), lambda i: (i, 0)),
        ],
        out_shape=[
            jax.ShapeDtypeStruct((N, LANES), jnp.float32),
            jax.ShapeDtypeStruct((N, 1), jnp.float32),
        ],
    )(cnt, h1)


def _tc_mid(s1, h1p, dinv, b1, W2):
    def body(s_ref, h_ref, d_ref, b_ref, w_ref, o_ref):
        t = s_ref[:, :D_H] + h_ref[:, :D_H]
        out1 = jnp.maximum(t * d_ref[...] + b_ref[...], 0.0)
        h2 = jnp.dot(out1, w_ref[...],
                     preferred_element_type=jnp.float32) * d_ref[...]
        o_ref[...] = jnp.concatenate([h2, jnp.zeros_like(h2)], axis=1)

    return pl.pallas_call(
        body,
        grid=(GB,),
        in_specs=[
            pl.BlockSpec((CB, LANES), lambda i: (i, 0)),
            pl.BlockSpec((CB, LANES), lambda i: (i, 0)),
            pl.BlockSpec((CB, 1), lambda i: (i, 0)),
            pl.BlockSpec((1, D_H), lambda i: (0, 0)),
            pl.BlockSpec((D_H, D_H), lambda i: (0, 0)),
        ],
        out_specs=pl.BlockSpec((CB, LANES), lambda i: (i, 0)),
        out_shape=jax.ShapeDtypeStruct((N, LANES), jnp.float32),
    )(s1, h1p, dinv, b1, W2)


def _tc_final(s2, h2p, dinv, b2, batch3, fc_W, fc_b):
    def body(s_ref, h_ref, d_ref, b_ref, bt_ref, w_ref, fb_ref, o_ref, acc):
        i = pl.program_id(0)

        @pl.when(i == 0)
        def _():
            acc[...] = jnp.zeros_like(acc)

        t = s_ref[:, :D_H] + h_ref[:, :D_H]
        out2 = t * d_ref[...] + b_ref[...]
        bt = bt_ref[0]
        gids = lax.broadcasted_iota(jnp.int32, (NG, CB), 0)
        onehot = (gids == bt).astype(jnp.float32)
        acc[...] += jnp.dot(onehot, out2, preferred_element_type=jnp.float32)

        @pl.when(i == GB - 1)
        def _():
            logits = jnp.dot(acc[...], w_ref[...],
                             preferred_element_type=jnp.float32) + fb_ref[...]
            m = jnp.max(logits, axis=1, keepdims=True)
            lse = jnp.log(jnp.sum(jnp.exp(logits - m), axis=1,
                                  keepdims=True)) + m
            o_ref[...] = logits - lse

    return pl.pallas_call(
        body,
        grid=(GB,),
        in_specs=[
            pl.BlockSpec((CB, LANES), lambda i: (i, 0)),
            pl.BlockSpec((CB, LANES), lambda i: (i, 0)),
            pl.BlockSpec((CB, 1), lambda i: (i, 0)),
            pl.BlockSpec((1, D_H), lambda i: (0, 0)),
            pl.BlockSpec((1, 1, CB), lambda i: (i, 0, 0)),
            pl.BlockSpec((D_H, NCLS), lambda i: (0, 0)),
            pl.BlockSpec((1, NCLS), lambda i: (0, 0)),
        ],
        out_specs=pl.BlockSpec((NG, NCLS), lambda i: (0, 0)),
        out_shape=jax.ShapeDtypeStruct((NG, NCLS), jnp.float32),
        scratch_shapes=[pltpu.VMEM((NG, D_H), jnp.float32)],
    )(s2, h2p, dinv, b2, batch3, fc_W, fc_b)


def kernel(x, edge_index, batch, W1, b1, W2, b2, fc_W, fc_b):
    src = edge_index[0].astype(jnp.int32)
    dst = edge_index[1].astype(jnp.int32)
    ones = jnp.ones((CH, 16), jnp.float32)
    zeros = jnp.zeros((N, LANES), jnp.float32)

    h1 = _tc_mm(x, W1)
    cnt = _sc_degree(dst, ones, zeros)
    h1p, dinv = _tc_scale(cnt, h1)
    s1 = _sc_scatter(h1p, src, dst, zeros)
    h2p = _tc_mid(s1, h1p, dinv, b1.reshape(1, D_H), W2)
    s2 = _sc_scatter(h2p, src, dst, zeros)
    return _tc_final(s2, h2p, dinv, b2.reshape(1, D_H),
                     batch.reshape(GB, 1, CB).astype(jnp.int32),
                     fc_W, fc_b.reshape(1, NCLS))
